# baseline (device time: 44462 ns/iter reference)
import jax
import jax.numpy as jnp
from jax import lax
from jax.experimental import pallas as pl
from jax.experimental.pallas import tpu as pltpu

N_DEV = 4
M = 1024
N = 1024
H = 256
Q = 128
CW = 128
NC = 8


def kernel(x):
    x2 = x.reshape(M, N)

    def body(x_ref, out_ref, bufA1, bufB1, bufA2, bufB2,
             sendA, recvA, sendB, recvB):
        p = lax.axis_index("i")
        yp = p ^ 1
        xp = 3 - p
        s1 = (p & 1) ^ (p >> 1)
        s2 = p >> 1
        s1b = p >> 1
        s2b = p & 1

        barrier = pltpu.get_barrier_semaphore()
        for nbr in (yp, xp):
            pl.semaphore_signal(
                barrier, inc=1,
                device_id=(nbr,), device_id_type=pl.DeviceIdType.MESH,
            )
        pl.semaphore_wait(barrier, 2)

        aK = H * s1
        aS = H * (1 - s1)
        bK = 512 + H * s1b
        bS = 512 + H * (1 - s1b)
        a2K = aK + Q * s2
        a2S = aK + Q * (1 - s2)
        b2K = bK + Q * s2b
        b2S = bK + Q * (1 - s2b)
        aFwd = Q * (1 - s2)
        aKeep = Q * s2
        bFwd = Q * (1 - s2b)
        bKeep = Q * s2b

        def rdma(src, dst, ssem, rsem, tgt):
            return pltpu.make_async_remote_copy(
                src_ref=src, dst_ref=dst, send_sem=ssem, recv_sem=rsem,
                device_id=(tgt,), device_id_type=pl.DeviceIdType.MESH)

        def make_col(c):
            cs = pl.ds(c * CW, CW)
            o = 6 * c
            d = {}
            d["r1a1"] = rdma(x_ref.at[pl.ds(aS + aFwd, Q), cs],
                             bufA1.at[pl.ds(aFwd, Q), cs],
                             sendA.at[o + 0], recvA.at[o + 0], yp)
            d["r1b1"] = rdma(x_ref.at[pl.ds(bS + bKeep, Q), cs],
                             bufB1.at[pl.ds(bKeep, Q), cs],
                             sendB.at[o + 0], recvB.at[o + 0], xp)
            d["r1a2"] = rdma(x_ref.at[pl.ds(aS + aKeep, Q), cs],
                             bufA1.at[pl.ds(aKeep, Q), cs],
                             sendA.at[o + 1], recvA.at[o + 1], yp)
            d["r1b2"] = rdma(x_ref.at[pl.ds(bS + bFwd, Q), cs],
                             bufB1.at[pl.ds(bFwd, Q), cs],
                             sendB.at[o + 1], recvB.at[o + 1], xp)
            d["r2a"] = rdma(out_ref.at[pl.ds(a2S, Q), cs],
                            bufA2.at[:, cs],
                            sendA.at[o + 2], recvA.at[o + 2], xp)
            d["r2b"] = rdma(out_ref.at[pl.ds(b2S, Q), cs],
                            bufB2.at[:, cs],
                            sendB.at[o + 2], recvB.at[o + 2], yp)
            d["r3a"] = rdma(out_ref.at[pl.ds(a2K, Q), cs],
                            out_ref.at[pl.ds(a2K, Q), cs],
                            sendA.at[o + 3], recvA.at[o + 3], xp)
            d["r3b"] = rdma(out_ref.at[pl.ds(b2K, Q), cs],
                            out_ref.at[pl.ds(b2K, Q), cs],
                            sendB.at[o + 3], recvB.at[o + 3], yp)
            d["r4a1"] = rdma(out_ref.at[pl.ds(a2K, Q), cs],
                             out_ref.at[pl.ds(a2K, Q), cs],
                             sendA.at[o + 4], recvA.at[o + 4], yp)
            d["r4b1"] = rdma(out_ref.at[pl.ds(b2K, Q), cs],
                             out_ref.at[pl.ds(b2K, Q), cs],
                             sendB.at[o + 4], recvB.at[o + 4], xp)
            d["r4a2"] = rdma(out_ref.at[pl.ds(a2S, Q), cs],
                             out_ref.at[pl.ds(a2S, Q), cs],
                             sendA.at[o + 5], recvA.at[o + 5], yp)
            d["r4b2"] = rdma(out_ref.at[pl.ds(b2S, Q), cs],
                             out_ref.at[pl.ds(b2S, Q), cs],
                             sendB.at[o + 5], recvB.at[o + 5], xp)
            return d

        cols = [make_col(c) for c in range(NC)]

        def cslice(c):
            return pl.ds(c * CW, CW)

        for c in range(NC):
            cols[c]["r1a1"].start()
            cols[c]["r1b1"].start()
        for c in range(NC):
            cols[c]["r1a2"].start()
            cols[c]["r1b2"].start()

        for c in range(NC):
            cs = cslice(c)
            cols[c]["r1a1"].wait_recv()
            out_ref[pl.ds(a2S, Q), cs] = x_ref[pl.ds(a2S, Q), cs] + \
                bufA1[pl.ds(aFwd, Q), cs]
            cols[c]["r2a"].start()
            cols[c]["r1b1"].wait_recv()
            out_ref[pl.ds(b2S, Q), cs] = x_ref[pl.ds(b2S, Q), cs] + \
                bufB1[pl.ds(bFwd, Q), cs]
            cols[c]["r2b"].start()

        for c in range(NC):
            cs = cslice(c)
            cols[c]["r1a2"].wait_recv()
            out_ref[pl.ds(a2K, Q), cs] = x_ref[pl.ds(a2K, Q), cs] + \
                bufA1[pl.ds(aKeep, Q), cs]
            cols[c]["r1b2"].wait_recv()
            out_ref[pl.ds(b2K, Q), cs] = x_ref[pl.ds(b2K, Q), cs] + \
                bufB1[pl.ds(bKeep, Q), cs]

        for c in range(NC):
            cs = cslice(c)
            cols[c]["r2a"].wait_recv()
            out_ref[pl.ds(a2K, Q), cs] = out_ref[pl.ds(a2K, Q), cs] + \
                bufA2[:, cs]
            cols[c]["r3a"].start()
            cols[c]["r4a1"].start()
            cols[c]["r2b"].wait_recv()
            out_ref[pl.ds(b2K, Q), cs] = out_ref[pl.ds(b2K, Q), cs] + \
                bufB2[:, cs]
            cols[c]["r3b"].start()
            cols[c]["r4b1"].start()

        for c in range(NC):
            cols[c]["r3a"].wait_recv()
            cols[c]["r4a2"].start()
            cols[c]["r3b"].wait_recv()
            cols[c]["r4b2"].start()

        for c in range(NC):
            cols[c]["r4a1"].wait_recv()
            cols[c]["r4a2"].wait_recv()
            cols[c]["r4b1"].wait_recv()
            cols[c]["r4b2"].wait_recv()

        for c in range(NC):
            for r in cols[c].values():
                r.wait_send()

    return pl.pallas_call(
        body,
        out_shape=jax.ShapeDtypeStruct((M, N), jnp.float32),
        in_specs=[pl.BlockSpec(memory_space=pltpu.VMEM)],
        out_specs=pl.BlockSpec(memory_space=pltpu.VMEM),
        scratch_shapes=[
            pltpu.VMEM((H, N), jnp.float32),
            pltpu.VMEM((H, N), jnp.float32),
            pltpu.VMEM((Q, N), jnp.float32),
            pltpu.VMEM((Q, N), jnp.float32),
            pltpu.SemaphoreType.DMA((48,)),
            pltpu.SemaphoreType.DMA((48,)),
            pltpu.SemaphoreType.DMA((48,)),
            pltpu.SemaphoreType.DMA((48,)),
        ],
        compiler_params=pltpu.CompilerParams(collective_id=0),
    )(x2)


# device time: 43415 ns/iter; 1.0241x vs baseline; 1.0241x over previous
import jax
import jax.numpy as jnp
from jax import lax
from jax.experimental import pallas as pl
from jax.experimental.pallas import tpu as pltpu

N_DEV = 4
M = 1024
N = 1024
H = 256
Q = 128
CW = 256
NC = 4
NS = 5


def kernel(x):
    x2 = x.reshape(M, N)

    def body(x_ref, out_ref, bufA1, bufB1, bufA2, bufB2,
             sendA, recvA, sendB, recvB):
        p = lax.axis_index("i")
        yp = p ^ 1
        xp = 3 - p
        s1 = (p & 1) ^ (p >> 1)
        s2 = p >> 1
        s1b = p >> 1
        s2b = p & 1

        barrier = pltpu.get_barrier_semaphore()
        for nbr in (yp, xp):
            pl.semaphore_signal(
                barrier, inc=1,
                device_id=(nbr,), device_id_type=pl.DeviceIdType.MESH,
            )
        pl.semaphore_wait(barrier, 2)

        aK = H * s1
        aS = H * (1 - s1)
        bK = 512 + H * s1b
        bS = 512 + H * (1 - s1b)
        a2K = aK + Q * s2
        a2S = aK + Q * (1 - s2)
        b2K = bK + Q * s2b
        b2S = bK + Q * (1 - s2b)
        aFwd = Q * (1 - s2)
        aKeep = Q * s2
        bFwd = Q * (1 - s2b)
        bKeep = Q * s2b

        def rdma(src, dst, ssem, rsem, tgt):
            return pltpu.make_async_remote_copy(
                src_ref=src, dst_ref=dst, send_sem=ssem, recv_sem=rsem,
                device_id=(tgt,), device_id_type=pl.DeviceIdType.MESH)

        def make_col(c):
            cs = pl.ds(c * CW, CW)
            o = NS * c
            d = {}
            d["r1a"] = rdma(x_ref.at[pl.ds(aS, H), cs],
                            bufA1.at[:, cs],
                            sendA.at[o + 0], recvA.at[o + 0], yp)
            d["r1b"] = rdma(x_ref.at[pl.ds(bS, H), cs],
                            bufB1.at[:, cs],
                            sendB.at[o + 0], recvB.at[o + 0], xp)
            d["r2a"] = rdma(out_ref.at[pl.ds(a2S, Q), cs],
                            bufA2.at[:, cs],
                            sendA.at[o + 1], recvA.at[o + 1], xp)
            d["r2b"] = rdma(out_ref.at[pl.ds(b2S, Q), cs],
                            bufB2.at[:, cs],
                            sendB.at[o + 1], recvB.at[o + 1], yp)
            d["r3a"] = rdma(out_ref.at[pl.ds(a2K, Q), cs],
                            out_ref.at[pl.ds(a2K, Q), cs],
                            sendA.at[o + 2], recvA.at[o + 2], xp)
            d["r3b"] = rdma(out_ref.at[pl.ds(b2K, Q), cs],
                            out_ref.at[pl.ds(b2K, Q), cs],
                            sendB.at[o + 2], recvB.at[o + 2], yp)
            d["r4a1"] = rdma(out_ref.at[pl.ds(a2K, Q), cs],
                             out_ref.at[pl.ds(a2K, Q), cs],
                             sendA.at[o + 3], recvA.at[o + 3], yp)
            d["r4b1"] = rdma(out_ref.at[pl.ds(b2K, Q), cs],
                             out_ref.at[pl.ds(b2K, Q), cs],
                             sendB.at[o + 3], recvB.at[o + 3], xp)
            d["r4a2"] = rdma(out_ref.at[pl.ds(a2S, Q), cs],
                             out_ref.at[pl.ds(a2S, Q), cs],
                             sendA.at[o + 4], recvA.at[o + 4], yp)
            d["r4b2"] = rdma(out_ref.at[pl.ds(b2S, Q), cs],
                             out_ref.at[pl.ds(b2S, Q), cs],
                             sendB.at[o + 4], recvB.at[o + 4], xp)
            return d

        cols = [make_col(c) for c in range(NC)]

        def cslice(c):
            return pl.ds(c * CW, CW)

        for c in range(NC):
            cols[c]["r1a"].start()
            cols[c]["r1b"].start()

        for c in range(NC):
            cs = cslice(c)
            cols[c]["r1a"].wait_recv()
            out_ref[pl.ds(a2S, Q), cs] = x_ref[pl.ds(a2S, Q), cs] + \
                bufA1[pl.ds(aFwd, Q), cs]
            cols[c]["r2a"].start()
            cols[c]["r1b"].wait_recv()
            out_ref[pl.ds(b2S, Q), cs] = x_ref[pl.ds(b2S, Q), cs] + \
                bufB1[pl.ds(bFwd, Q), cs]
            cols[c]["r2b"].start()

        for c in range(NC):
            cs = cslice(c)
            cols[c]["r2a"].wait_recv()
            out_ref[pl.ds(a2K, Q), cs] = x_ref[pl.ds(a2K, Q), cs] + \
                bufA1[pl.ds(aKeep, Q), cs] + bufA2[:, cs]
            cols[c]["r3a"].start()
            cols[c]["r4a1"].start()
            cols[c]["r2b"].wait_recv()
            out_ref[pl.ds(b2K, Q), cs] = x_ref[pl.ds(b2K, Q), cs] + \
                bufB1[pl.ds(bKeep, Q), cs] + bufB2[:, cs]
            cols[c]["r3b"].start()
            cols[c]["r4b1"].start()

        for c in range(NC):
            cols[c]["r3a"].wait_recv()
            cols[c]["r4a2"].start()
            cols[c]["r3b"].wait_recv()
            cols[c]["r4b2"].start()

        for c in range(NC):
            cols[c]["r4a1"].wait_recv()
            cols[c]["r4a2"].wait_recv()
            cols[c]["r4b1"].wait_recv()
            cols[c]["r4b2"].wait_recv()

        for c in range(NC):
            for r in cols[c].values():
                r.wait_send()

    return pl.pallas_call(
        body,
        out_shape=jax.ShapeDtypeStruct((M, N), jnp.float32),
        in_specs=[pl.BlockSpec(memory_space=pltpu.VMEM)],
        out_specs=pl.BlockSpec(memory_space=pltpu.VMEM),
        scratch_shapes=[
            pltpu.VMEM((H, N), jnp.float32),
            pltpu.VMEM((H, N), jnp.float32),
            pltpu.VMEM((Q, N), jnp.float32),
            pltpu.VMEM((Q, N), jnp.float32),
            pltpu.SemaphoreType.DMA((NS * NC,)),
            pltpu.SemaphoreType.DMA((NS * NC,)),
            pltpu.SemaphoreType.DMA((NS * NC,)),
            pltpu.SemaphoreType.DMA((NS * NC,)),
        ],
        compiler_params=pltpu.CompilerParams(collective_id=0),
    )(x2)


# device time: 41954 ns/iter; 1.0598x vs baseline; 1.0348x over previous
import jax
import jax.numpy as jnp
from jax import lax
from jax.experimental import pallas as pl
from jax.experimental.pallas import tpu as pltpu

N_DEV = 4
M = 1024
N = 1024
H = 256
CW = 256
NC = 4
NS = 3


def kernel(x):
    x2 = x.reshape(M, N)

    def body(x_ref, out_ref, bufA1, bufB1, bufA2, bufB2,
             sendA, recvA, sendB, recvB):
        p = lax.axis_index("i")
        yp = p ^ 1
        xp = 3 - p
        s1 = (p & 1) ^ (p >> 1)
        s1b = p >> 1

        barrier = pltpu.get_barrier_semaphore()
        for nbr in (yp, xp):
            pl.semaphore_signal(
                barrier, inc=1,
                device_id=(nbr,), device_id_type=pl.DeviceIdType.MESH,
            )
        pl.semaphore_wait(barrier, 2)

        aK = H * s1
        aS = H * (1 - s1)
        bK = 512 + H * s1b
        bS = 512 + H * (1 - s1b)

        def rdma(src, dst, ssem, rsem, tgt):
            return pltpu.make_async_remote_copy(
                src_ref=src, dst_ref=dst, send_sem=ssem, recv_sem=rsem,
                device_id=(tgt,), device_id_type=pl.DeviceIdType.MESH)

        def make_col(c):
            cs = pl.ds(c * CW, CW)
            o = NS * c
            d = {}
            d["r1a"] = rdma(x_ref.at[pl.ds(aS, H), cs], bufA1.at[:, cs],
                            sendA.at[o + 0], recvA.at[o + 0], yp)
            d["r1b"] = rdma(x_ref.at[pl.ds(bS, H), cs], bufB1.at[:, cs],
                            sendB.at[o + 0], recvB.at[o + 0], xp)
            d["r2a"] = rdma(out_ref.at[pl.ds(aK, H), cs], bufA2.at[:, cs],
                            sendA.at[o + 1], recvA.at[o + 1], xp)
            d["r2b"] = rdma(out_ref.at[pl.ds(bK, H), cs], bufB2.at[:, cs],
                            sendB.at[o + 1], recvB.at[o + 1], yp)
            d["r3a"] = rdma(out_ref.at[pl.ds(aK, H), cs],
                            out_ref.at[pl.ds(aK, H), cs],
                            sendA.at[o + 2], recvA.at[o + 2], yp)
            d["r3b"] = rdma(out_ref.at[pl.ds(bK, H), cs],
                            out_ref.at[pl.ds(bK, H), cs],
                            sendB.at[o + 2], recvB.at[o + 2], xp)
            return d

        cols = [make_col(c) for c in range(NC)]

        def cslice(c):
            return pl.ds(c * CW, CW)

        for c in range(NC):
            cols[c]["r1a"].start()
            cols[c]["r1b"].start()

        for c in range(NC):
            cs = cslice(c)
            cols[c]["r1a"].wait_recv()
            out_ref[pl.ds(aK, H), cs] = x_ref[pl.ds(aK, H), cs] + \
                bufA1[:, cs]
            cols[c]["r2a"].start()
            cols[c]["r1b"].wait_recv()
            out_ref[pl.ds(bK, H), cs] = x_ref[pl.ds(bK, H), cs] + \
                bufB1[:, cs]
            cols[c]["r2b"].start()

        for c in range(NC):
            cs = cslice(c)
            cols[c]["r2a"].wait_send()
            cols[c]["r2a"].wait_recv()
            out_ref[pl.ds(aK, H), cs] = out_ref[pl.ds(aK, H), cs] + \
                bufA2[:, cs]
            cols[c]["r3a"].start()
            cols[c]["r2b"].wait_send()
            cols[c]["r2b"].wait_recv()
            out_ref[pl.ds(bK, H), cs] = out_ref[pl.ds(bK, H), cs] + \
                bufB2[:, cs]
            cols[c]["r3b"].start()

        for c in range(NC):
            cols[c]["r3a"].wait_recv()
            cols[c]["r3b"].wait_recv()

        for c in range(NC):
            for k in ("r1a", "r1b", "r3a", "r3b"):
                cols[c][k].wait_send()

    return pl.pallas_call(
        body,
        out_shape=jax.ShapeDtypeStruct((M, N), jnp.float32),
        in_specs=[pl.BlockSpec(memory_space=pltpu.VMEM)],
        out_specs=pl.BlockSpec(memory_space=pltpu.VMEM),
        scratch_shapes=[
            pltpu.VMEM((H, N), jnp.float32),
            pltpu.VMEM((H, N), jnp.float32),
            pltpu.VMEM((H, N), jnp.float32),
            pltpu.VMEM((H, N), jnp.float32),
            pltpu.SemaphoreType.DMA((NS * NC,)),
            pltpu.SemaphoreType.DMA((NS * NC,)),
            pltpu.SemaphoreType.DMA((NS * NC,)),
            pltpu.SemaphoreType.DMA((NS * NC,)),
        ],
        compiler_params=pltpu.CompilerParams(collective_id=0),
    )(x2)
